# baseline (speedup 1.0000x reference)
import functools

import jax
import jax.numpy as jnp
from jax import lax
from jax.experimental import pallas as pl
from jax.experimental.pallas import tpu as pltpu
from jax.experimental.pallas import tpu_sc as plsc

NC, NS = 2, 16
NW = NC * NS
E = 320000
T = 160000
D = 128

NLOC = 5120
NF_ROWS = NC * NLOC

R = 8064
NPASS = 20
V_PAD = NC * R * NPASS

_MESH = plsc.VectorSubcoreMesh(core_axis_name="c", subcore_axis_name="s",
                               num_cores=NC, num_subcores=NS)

_SC_PARAMS = pltpu.CompilerParams(needs_layout_passes=False)

KD = 2000


def _foreach_block(rows, dtype, fn):
    bc = 32 if dtype == jnp.bfloat16 else 16

    @pl.loop(0, rows)
    def _(r):
        @pl.loop(0, D // bc)
        def _(j):
            fn((r, pl.ds(j * bc, bc)))


def _zero_rows(ref, rows, dtype):
    bc = 32 if dtype == jnp.bfloat16 else 16
    z = jnp.zeros((bc,), dtype)

    def fn(slc):
        ref[slc] = z

    _foreach_block(rows, dtype, fn)


def _negate_rows(ref, rows, dtype):
    def fn(slc):
        ref[slc] = -ref[slc]

    _foreach_block(rows, dtype, fn)


def _scan_compact_accumulate(streams, table_hbm, acc_sh, lo, rng, seg_len,
                             sbase, ebuf, cel, ctr, elrow, gbuf, dtype):
    iota16 = lax.iota(jnp.int32, 16)
    zeros16i = jnp.zeros((16,), jnp.int32)

    for eh, sign_neg in streams:
        def vec_body(i, cnt, _k, _lo):
            e = ebuf[pl.ds(i * 16, 16)]
            el = e - _lo
            m = (el >= 0) & (el < rng)
            pos = sbase + _k * KD + i * 16 + iota16
            plsc.store_compressed(cel.at[pl.ds(cnt, 16)], el, mask=m)
            plsc.store_compressed(ctr.at[pl.ds(cnt, 16)], pos, mask=m)
            pc = plsc.all_reduce_population_count(m)
            return cnt + jnp.max(pc, axis=0)

        def scan_chunk(k, cnt, _eh, _lo):
            pltpu.sync_copy(_eh.at[pl.ds(sbase + k * KD, KD)], ebuf)
            return lax.fori_loop(0, KD // 16,
                                 functools.partial(vec_body, _k=k, _lo=_lo),
                                 cnt)

        cnt = lax.fori_loop(0, seg_len // KD,
                            functools.partial(scan_chunk, _eh=eh, _lo=lo),
                            jnp.int32(0))

        for j in range(8):
            cel[pl.ds(cnt + j * 16, 16)] = rng + iota16
            ctr[pl.ds(cnt + j * 16, 16)] = zeros16i

        nb = (cnt + 127) // 128

        def batch_body(g, carry, _neg):
            pltpu.sync_copy(table_hbm.at[ctr.at[pl.ds(g * 128, 128)]], gbuf)
            if _neg:
                _negate_rows(gbuf, 128, dtype)
            for j in range(8):
                elrow[pl.ds(j * 16, 16)] = cel[pl.ds(g * 128 + j * 16, 16)]
            pltpu.sync_copy(gbuf, acc_sh.at[elrow], add=True)
            return carry

        lax.fori_loop(0, nb,
                      functools.partial(batch_body, _neg=sign_neg),
                      jnp.int32(0))


def _node_scatter(x, src, dst):
    seg = E // NS

    @functools.partial(
        pl.kernel,
        out_type=jax.ShapeDtypeStruct((NF_ROWS, D), jnp.float32),
        mesh=_MESH,
        compiler_params=_SC_PARAMS,
        scratch_types=[
            pltpu.VMEM((KD,), jnp.int32),
            pltpu.VMEM((seg + 256,), jnp.int32),
            pltpu.VMEM((seg + 256,), jnp.int32),
            pltpu.VMEM((128,), jnp.int32),
            pltpu.VMEM((128, D), jnp.float32),
            pltpu.VMEM((128, D), jnp.float32),
            pltpu.VMEM_SHARED((NLOC + 16, D), jnp.float32),
        ])
    def body(x_hbm, src_hbm, dst_hbm, nf_hbm,
             ebuf, cel, ctr, elrow, gbuf, zbuf, nfsh):
        c = lax.axis_index("c")
        s = lax.axis_index("s")
        _zero_rows(zbuf, 128, jnp.float32)
        span = NLOC // NS
        for off in range(0, span, 128):
            sz = min(128, span - off)
            pltpu.sync_copy(zbuf.at[pl.ds(0, sz)],
                            nfsh.at[pl.ds(s * span + off, sz)])
        plsc.subcore_barrier()
        _scan_compact_accumulate(
            ((dst_hbm, False), (src_hbm, True)),
            x_hbm, nfsh, c * NLOC, NLOC, seg, s * seg,
            ebuf, cel, ctr, elrow, gbuf, jnp.float32)
        plsc.subcore_barrier()
        pltpu.sync_copy(nfsh.at[pl.ds(s * span, span)],
                        nf_hbm.at[pl.ds(c * NLOC + s * span, span)])

    return body(x, src, dst)


def _node_gather(nf, src, dst):
    @functools.partial(
        pl.kernel,
        out_type=jax.ShapeDtypeStruct((E, D), jnp.float32),
        mesh=_MESH,
        scratch_types=[
            pltpu.VMEM((256,), jnp.int32),
            pltpu.VMEM((256,), jnp.int32),
            pltpu.VMEM((256, D), jnp.float32),
            pltpu.VMEM((256, D), jnp.float32),
        ])
    def body(nf_hbm, src_hbm, dst_hbm, u_hbm, sidx, didx, gd, gs):
        w = lax.axis_index("c") * NS + lax.axis_index("s")

        @pl.loop(0, 40)
        def _(k):
            cid = w + k * NW

            @pl.when(cid < E // 256)
            def _():
                pltpu.sync_copy(src_hbm.at[pl.ds(cid * 256, 256)], sidx)
                pltpu.sync_copy(dst_hbm.at[pl.ds(cid * 256, 256)], didx)
                for b in range(2):
                    sl = pl.ds(b * 128, 128)
                    pltpu.sync_copy(nf_hbm.at[didx.at[sl]], gd.at[sl])
                    pltpu.sync_copy(nf_hbm.at[sidx.at[sl]], gs.at[sl])

                def sub_fn(slc):
                    gd[slc] = gd[slc] - gs[slc]

                _foreach_block(256, jnp.float32, sub_fn)
                pltpu.sync_copy(gd, u_hbm.at[pl.ds(cid * 256, 256)])

    return body(nf, src, dst)


def _tri_gather(x, e0, e1, e2):
    @functools.partial(
        pl.kernel,
        out_type=jax.ShapeDtypeStruct((T, D), jnp.float32),
        mesh=_MESH,
        scratch_types=[
            pltpu.VMEM((256,), jnp.int32),
            pltpu.VMEM((256,), jnp.int32),
            pltpu.VMEM((256,), jnp.int32),
            pltpu.VMEM((256, D), jnp.float32),
            pltpu.VMEM((256, D), jnp.float32),
            pltpu.VMEM((256, D), jnp.float32),
        ])
    def body(x_hbm, e0_hbm, e1_hbm, e2_hbm, tf_hbm,
             i0, i1, i2, g0, g1, g2):
        w = lax.axis_index("c") * NS + lax.axis_index("s")

        @pl.loop(0, 20)
        def _(k):
            cid = w + k * NW

            @pl.when(cid < T // 256)
            def _():
                pltpu.sync_copy(e0_hbm.at[pl.ds(cid * 256, 256)], i0)
                pltpu.sync_copy(e1_hbm.at[pl.ds(cid * 256, 256)], i1)
                pltpu.sync_copy(e2_hbm.at[pl.ds(cid * 256, 256)], i2)
                for b in range(2):
                    sl = pl.ds(b * 128, 128)
                    pltpu.sync_copy(x_hbm.at[i0.at[sl]], g0.at[sl])
                    pltpu.sync_copy(x_hbm.at[i1.at[sl]], g1.at[sl])
                    pltpu.sync_copy(x_hbm.at[i2.at[sl]], g2.at[sl])

                def comb_fn(slc):
                    g0[slc] = g0[slc] - g1[slc] + g2[slc]

                _foreach_block(256, jnp.float32, comb_fn)
                pltpu.sync_copy(g0, tf_hbm.at[pl.ds(cid * 256, 256)])

    return body(x, e0, e1, e2)


def _tri_scatter(tf, e0, e1, e2):
    seg = T // NS

    @functools.partial(
        pl.kernel,
        out_type=jax.ShapeDtypeStruct((V_PAD, D), jnp.float32),
        mesh=_MESH,
        compiler_params=_SC_PARAMS,
        scratch_types=[
            pltpu.VMEM((KD,), jnp.int32),
            pltpu.VMEM((seg + 256,), jnp.int32),
            pltpu.VMEM((seg + 256,), jnp.int32),
            pltpu.VMEM((128,), jnp.int32),
            pltpu.VMEM((128, D), jnp.float32),
            pltpu.VMEM((128, D), jnp.float32),
            pltpu.VMEM_SHARED((R + 16, D), jnp.float32),
        ])
    def body(tf_hbm, e0_hbm, e1_hbm, e2_hbm, v_hbm,
             ebuf, cel, ctr, elrow, gbuf, zbuf, vsh):
        c = lax.axis_index("c")
        s = lax.axis_index("s")
        _zero_rows(zbuf, 128, jnp.float32)
        span = R // NS

        @pl.loop(0, NPASS)
        def _(p):
            lo = (NC * p + c) * R
            for off in range(0, span, 128):
                sz = min(128, span - off)
                pltpu.sync_copy(zbuf.at[pl.ds(0, sz)],
                                vsh.at[pl.ds(s * span + off, sz)])
            plsc.subcore_barrier()
            _scan_compact_accumulate(
                ((e0_hbm, False), (e1_hbm, True), (e2_hbm, False)),
                tf_hbm, vsh, lo, R, seg, s * seg,
                ebuf, cel, ctr, elrow, gbuf, jnp.float32)
            plsc.subcore_barrier()
            pltpu.sync_copy(vsh.at[pl.ds(s * span, span)],
                            v_hbm.at[pl.ds(lo + s * span, span)])
            plsc.subcore_barrier()

    return body(tf, e0, e1, e2)


def _cast_bf16(a, block_rows):
    def cast_body(a_ref, o_ref):
        o_ref[...] = a_ref[...].astype(jnp.bfloat16)

    n = a.shape[0]
    return pl.pallas_call(
        cast_body,
        grid=(n // block_rows,),
        in_specs=[pl.BlockSpec((block_rows, D), lambda i: (i, 0))],
        out_specs=pl.BlockSpec((block_rows, D), lambda i: (i, 0)),
        out_shape=jax.ShapeDtypeStruct((n, D), jnp.bfloat16),
    )(a)


def _combine(u, v, x, W0, W1, W2):
    def combine_body(u_ref, v_ref, x_ref, w0_ref, w1_ref, w2_ref, o_ref):
        acc = jnp.dot(u_ref[...], w0_ref[...],
                      preferred_element_type=jnp.float32)
        acc += jnp.dot(v_ref[...], w1_ref[...],
                       preferred_element_type=jnp.float32)
        acc += jnp.dot(x_ref[...], w2_ref[...],
                       preferred_element_type=jnp.float32)
        o_ref[...] = jnp.tanh(acc)

    B = 512
    return pl.pallas_call(
        combine_body,
        grid=(E // B,),
        in_specs=[pl.BlockSpec((B, D), lambda i: (i, 0))] * 3
        + [pl.BlockSpec((D, D), lambda i: (0, 0))] * 3,
        out_specs=pl.BlockSpec((B, D), lambda i: (i, 0)),
        out_shape=jax.ShapeDtypeStruct((E, D), jnp.float32),
    )(u, v, x, W0, W1, W2)


def kernel(x, edge_index, tri_edges, W0, W1, W2):
    src = edge_index[0].astype(jnp.int32)
    dst = edge_index[1].astype(jnp.int32)
    e0 = tri_edges[0].astype(jnp.int32)
    e1 = tri_edges[1].astype(jnp.int32)
    e2 = tri_edges[2].astype(jnp.int32)

    nf = _node_scatter(x, src, dst)
    u = _node_gather(nf, src, dst)
    tf = _tri_gather(x, e0, e1, e2)
    v = _tri_scatter(tf, e0, e1, e2)[:E]
    return _combine(u, v, x, W0, W1, W2)
